# Initial kernel scaffold; baseline (speedup 1.0000x reference)
import jax
import jax.numpy as jnp
from jax.experimental import pallas as pl


def kernel(positions, W):
    raise NotImplementedError("write your pallas kernel here")


# speedup vs baseline: 1.2361x; 1.2361x over previous
import functools
import jax
import jax.numpy as jnp
from jax import lax
from jax.experimental import pallas as pl
from jax.experimental.pallas import tpu as pltpu
from jax.experimental.pallas import tpu_sc as plsc

MAXP = 24
DM = 384
NPTS = 576
DHALF = DM // 2
NROWS = NPTS * NPTS * 2

NC = 2
NS = 16
NW = NC * NS
L = 16

I_PER_W = NPTS // NW
ROWS_PER_I = NPTS * 2
ROWS_PER_W = I_PER_W * ROWS_PER_I
CHUNK = 256
NCHUNK = ROWS_PER_W // CHUNK
JG = NPTS // L


def _body(pos_hbm, w_hbm, out_hbm, pos_v, idx_v, rows_v, sem, csem):
    wid = lax.axis_index("s") * NC + lax.axis_index("c")

    pltpu.sync_copy(pos_hbm, pos_v)

    iota = lax.iota(jnp.int32, L)

    def i_body(il, _):
        i = wid * I_PER_W + il
        ivx = jnp.full((L,), 2 * i, jnp.int32)
        pxi = plsc.load_gather(pos_v, [ivx])
        pyi = plsc.load_gather(pos_v, [ivx + 1])

        def j_body(jg, _):
            jv2 = 2 * (jg * L + iota)
            pxj = plsc.load_gather(pos_v, [jv2])
            pyj = plsc.load_gather(pos_v, [jv2 + 1])
            ix = jnp.clip(pxi - pxj, -MAXP, MAXP) + MAXP
            iy = jnp.clip(pyi - pyj, -MAXP, MAXP) + MAXP
            addr = il * ROWS_PER_I + jg * (2 * L) + 2 * iota
            plsc.store_scatter(idx_v, [addr], ix)
            plsc.store_scatter(idx_v, [addr + 1], iy)
            return 0

        lax.fori_loop(0, JG, j_body, 0)
        return 0

    lax.fori_loop(0, I_PER_W, i_body, 0)

    out_base = wid * ROWS_PER_W

    def c_body(c, _):
        pltpu.async_copy(
            w_hbm.at[idx_v.at[pl.ds(c * CHUNK, CHUNK)]], rows_v, sem
        ).wait()
        pltpu.sync_copy(rows_v, out_hbm.at[pl.ds(out_base + c * CHUNK, CHUNK)])
        return 0

    lax.fori_loop(0, NCHUNK, c_body, 0)


@jax.jit
def _run(positions, W):
    mesh = plsc.VectorSubcoreMesh(core_axis_name="c", subcore_axis_name="s")
    k = functools.partial(
        pl.kernel,
        mesh=mesh,
        compiler_params=pltpu.CompilerParams(
            needs_layout_passes=False, use_tc_tiling_on_sc=False
        ),
        out_type=jax.ShapeDtypeStruct((NROWS, DHALF), jnp.float32),
        scratch_types=[
            pltpu.VMEM((NPTS * 2,), jnp.int32),
            pltpu.VMEM((ROWS_PER_W,), jnp.int32),
            pltpu.VMEM((CHUNK, DHALF), jnp.float32),
            pltpu.SemaphoreType.DMA,
            pltpu.SemaphoreType.DMA,
        ],
    )(_body)
    return k(positions, W)


def kernel(positions, W):
    out = _run(positions.astype(jnp.int32).reshape(-1), W)
    return out.reshape(NPTS, NPTS, DM)


# speedup vs baseline: 1.2397x; 1.0029x over previous
import functools
import jax
import jax.numpy as jnp
from jax import lax
from jax.experimental import pallas as pl
from jax.experimental.pallas import tpu as pltpu
from jax.experimental.pallas import tpu_sc as plsc

MAXP = 24
DM = 384
NPTS = 576
DHALF = DM // 2
NROWS = NPTS * NPTS * 2

NC = 2
NS = 16
NW = NC * NS
L = 16

I_PER_W = NPTS // NW
ROWS_PER_I = NPTS * 2
ROWS_PER_W = I_PER_W * ROWS_PER_I
CHUNK = 128
NCHUNK = ROWS_PER_W // CHUNK
NBUF = 3
JG = NPTS // L


def _body(pos_hbm, w_hbm, out_hbm, pos_v, idx_v, rows_v, *sems):
    sem_g = sems[:NBUF]
    sem_w = sems[NBUF:]
    wid = lax.axis_index("s") * NC + lax.axis_index("c")

    pltpu.sync_copy(pos_hbm, pos_v)

    iota = lax.iota(jnp.int32, L)

    def i_body(il, _):
        i = wid * I_PER_W + il
        ivx = jnp.full((L,), 2 * i, jnp.int32)
        pxi = plsc.load_gather(pos_v, [ivx])
        pyi = plsc.load_gather(pos_v, [ivx + 1])

        def j_body(jg, _):
            jv2 = 2 * (jg * L + iota)
            pxj = plsc.load_gather(pos_v, [jv2])
            pyj = plsc.load_gather(pos_v, [jv2 + 1])
            ix = jnp.clip(pxi - pxj, -MAXP, MAXP) + MAXP
            iy = jnp.clip(pyi - pyj, -MAXP, MAXP) + MAXP
            addr = il * ROWS_PER_I + jg * (2 * L) + 2 * iota
            plsc.store_scatter(idx_v, [addr], ix)
            plsc.store_scatter(idx_v, [addr + 1], iy)
            return 0

        lax.fori_loop(0, JG, j_body, 0)
        return 0

    lax.fori_loop(0, I_PER_W, i_body, 0)

    out_base = wid * ROWS_PER_W

    def start_gather(c, b):
        pltpu.async_copy(
            w_hbm.at[idx_v.at[pl.ds(c * CHUNK, CHUNK)]],
            rows_v.at[b],
            sem_g[b],
        )

    for b in range(NBUF):
        start_gather(b, b)

    def g_body(g, _):
        for b in range(NBUF):
            c = g * NBUF + b
            pltpu.make_async_copy(
                w_hbm.at[idx_v.at[pl.ds(c * CHUNK, CHUNK)]],
                rows_v.at[b],
                sem_g[b],
            ).wait()
            out_sl = out_hbm.at[pl.ds(out_base + c * CHUNK, CHUNK)]
            pltpu.async_copy(rows_v.at[b], out_sl, sem_w[b])

            @pl.when(c + NBUF < NCHUNK)
            def _():
                pltpu.make_async_copy(rows_v.at[b], out_sl, sem_w[b]).wait()
                start_gather(c + NBUF, b)

        return 0

    lax.fori_loop(0, NCHUNK // NBUF, g_body, 0)

    for b in range(NBUF):
        c = NCHUNK - NBUF + b
        pltpu.make_async_copy(
            rows_v.at[b],
            out_hbm.at[pl.ds(out_base + c * CHUNK, CHUNK)],
            sem_w[b],
        ).wait()


@jax.jit
def _run(positions, W):
    mesh = plsc.VectorSubcoreMesh(core_axis_name="c", subcore_axis_name="s")
    k = functools.partial(
        pl.kernel,
        mesh=mesh,
        compiler_params=pltpu.CompilerParams(
            needs_layout_passes=False, use_tc_tiling_on_sc=False
        ),
        out_type=jax.ShapeDtypeStruct((NROWS, DHALF), jnp.float32),
        scratch_types=[
            pltpu.VMEM((NPTS * 2,), jnp.int32),
            pltpu.VMEM((ROWS_PER_W,), jnp.int32),
            pltpu.VMEM((NBUF, CHUNK, DHALF), jnp.float32),
        ] + [pltpu.SemaphoreType.DMA] * (2 * NBUF),
    )(_body)
    return k(positions, W)


def kernel(positions, W):
    out = _run(positions.astype(jnp.int32).reshape(-1), W)
    return out.reshape(NPTS, NPTS, DM)
